# baseline (speedup 1.0000x reference)
import functools

import jax
import jax.numpy as jnp
from jax import lax
from jax.experimental import pallas as pl
from jax.experimental.pallas import tpu as pltpu
from jax.experimental.pallas import tpu_sc as plsc

S, B, D = 250, 128, 1024
NC, NS = 2, 16
NW = NC * NS
UMAX = (S + NW - 1) // NW
CROWS = 16
CPU_ = B // CROWS
NCK_MAX = UMAX * CPU_
NSLICE = D // 16
NBUF = 6
LOOK = 4


def _chunk_slab(ref, wid, kk):
    s = wid * UMAX + kk // CPU_
    b0 = (kk % CPU_) * CROWS
    return ref.at[s, pl.ds(b0, CROWS), :]


def _sc_body(x_hbm, pe_hbm, out_hbm, pe_v, b0_v, b1_v, b2_v, b3_v, b4_v,
             b5_v, pe_sem, si0, si1, si2, si3, si4, si5, so0, so1, so2,
             so3, so4, so5):
    wid = lax.axis_index("s") * NC + lax.axis_index("c")
    bufs = (b0_v, b1_v, b2_v, b3_v, b4_v, b5_v)
    sin = (si0, si1, si2, si3, si4, si5)
    sout = (so0, so1, so2, so3, so4, so5)
    nck = jnp.where(wid < S // UMAX, NCK_MAX, (S - (S // UMAX) * UMAX) * CPU_)

    pltpu.async_copy(pe_hbm.at[pl.ds(wid * UMAX, 1), :],
                     pe_v.at[pl.ds(0, 1), :], pe_sem)
    for b in range(LOOK):
        pltpu.async_copy(_chunk_slab(x_hbm, wid, b), bufs[b], sin[b])

    def slot(g, _):
        for b in range(NBUF):
            kk = g * NBUF + b
            buf = bufs[b]
            bl = (b + LOOK) % NBUF

            @pl.when(kk < nck)
            def _():
                @pl.when(kk % CPU_ == 0)
                def _():
                    u = kk // CPU_
                    p = u % 2
                    pltpu.make_async_copy(
                        pe_hbm.at[pl.ds(wid, 1), :],
                        pe_v.at[pl.ds(0, 1), :], pe_sem).wait()

                    @pl.when(kk + CPU_ < nck)
                    def _():
                        for q in range(2):
                            @pl.when(p == q)
                            def _(q=q):
                                pltpu.async_copy(
                                    pe_hbm.at[
                                        pl.ds(wid * UMAX + u + 1, 1), :],
                                    pe_v.at[pl.ds(1 - q, 1), :], pe_sem)

                pltpu.make_async_copy(
                    _chunk_slab(x_hbm, wid, kk), buf, sin[b]).wait()

                @pl.when(kk + LOOK < nck)
                def _():
                    @pl.when(kk + LOOK >= NBUF)
                    def _():
                        pltpu.make_async_copy(
                            bufs[bl],
                            _chunk_slab(out_hbm, wid, kk + LOOK - NBUF),
                            sout[bl]).wait()

                    pltpu.async_copy(
                        _chunk_slab(x_hbm, wid, kk + LOOK), bufs[bl],
                        sin[bl])

                pp = (kk // CPU_) % 2

                def jstep(j, _):
                    sl = pl.ds(j * 16, 16)
                    pe_vec = pe_v[pp, sl]
                    for r in range(CROWS):
                        plsc.addupdate(buf.at[r, sl], pe_vec)
                    return 0

                lax.fori_loop(0, NSLICE, jstep, 0)

                pltpu.async_copy(buf, _chunk_slab(out_hbm, wid, kk),
                                 sout[b])

        return 0

    lax.fori_loop(0, NCK_MAX // NBUF + 1, slot, 0)

    for b in range(NBUF):
        kk_b = nck - NBUF + ((b - nck) % NBUF + NBUF) % NBUF
        pltpu.make_async_copy(
            bufs[b], _chunk_slab(out_hbm, wid, kk_b), sout[b]).wait()


def kernel(x, pos_embed):
    mesh = plsc.VectorSubcoreMesh(core_axis_name="c", subcore_axis_name="s")
    k = functools.partial(
        pl.kernel,
        mesh=mesh,
        out_type=jax.ShapeDtypeStruct((S, B, D), jnp.float32),
        scratch_types=(
            [pltpu.VMEM((2, D), jnp.float32)]
            + [pltpu.VMEM((CROWS, D), jnp.float32) for _ in range(NBUF)]
            + [pltpu.SemaphoreType.DMA for _ in range(2 * NBUF + 1)]
        ),
    )(_sc_body)
    return k(x, pos_embed)


# speedup vs baseline: 1.0028x; 1.0028x over previous
import functools

import jax
import jax.numpy as jnp
from jax import lax
from jax.experimental import pallas as pl
from jax.experimental.pallas import tpu as pltpu
from jax.experimental.pallas import tpu_sc as plsc

S, B, D = 250, 128, 1024
NC, NS = 2, 16
NW = NC * NS
UMAX = (S + NW - 1) // NW
CROWS = 16
CPU_ = B // CROWS
NCK_MAX = UMAX * CPU_
NSLICE = D // 16
NBUF = 6
LOOK = 4


def _chunk_slab(ref, wid, kk):
    s = wid + (kk // CPU_) * NW
    b0 = (kk % CPU_) * CROWS
    return ref.at[s, pl.ds(b0, CROWS), :]


def _sc_body(x_hbm, pe_hbm, out_hbm, pe_v, b0_v, b1_v, b2_v, b3_v, b4_v,
             b5_v, pe_sem, si0, si1, si2, si3, si4, si5, so0, so1, so2,
             so3, so4, so5):
    wid = lax.axis_index("s") * NC + lax.axis_index("c")
    bufs = (b0_v, b1_v, b2_v, b3_v, b4_v, b5_v)
    sin = (si0, si1, si2, si3, si4, si5)
    sout = (so0, so1, so2, so3, so4, so5)
    nck = jnp.where(wid < S - (UMAX - 1) * NW, NCK_MAX, NCK_MAX - CPU_)

    pltpu.async_copy(pe_hbm.at[pl.ds(wid, 1), :],
                     pe_v.at[pl.ds(0, 1), :], pe_sem)
    for b in range(LOOK):
        pltpu.async_copy(_chunk_slab(x_hbm, wid, b), bufs[b], sin[b])

    def slot(g, _):
        for b in range(NBUF):
            kk = g * NBUF + b
            buf = bufs[b]
            bl = (b + LOOK) % NBUF

            @pl.when(kk < nck)
            def _():
                @pl.when(kk % CPU_ == 0)
                def _():
                    u = kk // CPU_
                    p = u % 2
                    pltpu.make_async_copy(
                        pe_hbm.at[pl.ds(wid, 1), :],
                        pe_v.at[pl.ds(0, 1), :], pe_sem).wait()

                    @pl.when(wid + (u + 1) * NW < S)
                    def _():
                        for q in range(2):
                            @pl.when(p == q)
                            def _(q=q):
                                pltpu.async_copy(
                                    pe_hbm.at[
                                        pl.ds(wid + (u + 1) * NW, 1), :],
                                    pe_v.at[pl.ds(1 - q, 1), :], pe_sem)

                pltpu.make_async_copy(
                    _chunk_slab(x_hbm, wid, kk), buf, sin[b]).wait()

                @pl.when(kk + LOOK < nck)
                def _():
                    @pl.when(kk + LOOK >= NBUF)
                    def _():
                        pltpu.make_async_copy(
                            bufs[bl],
                            _chunk_slab(out_hbm, wid, kk + LOOK - NBUF),
                            sout[bl]).wait()

                    pltpu.async_copy(
                        _chunk_slab(x_hbm, wid, kk + LOOK), bufs[bl],
                        sin[bl])

                pp = (kk // CPU_) % 2

                def jstep(j, _):
                    sl = pl.ds(j * 16, 16)
                    pe_vec = pe_v[pp, sl]
                    for r in range(CROWS):
                        plsc.addupdate(buf.at[r, sl], pe_vec)
                    return 0

                lax.fori_loop(0, NSLICE, jstep, 0)

                pltpu.async_copy(buf, _chunk_slab(out_hbm, wid, kk),
                                 sout[b])

        return 0

    lax.fori_loop(0, NCK_MAX // NBUF + 1, slot, 0)

    for b in range(NBUF):
        kk_b = nck - NBUF + ((b - nck) % NBUF + NBUF) % NBUF
        pltpu.make_async_copy(
            bufs[b], _chunk_slab(out_hbm, wid, kk_b), sout[b]).wait()


def kernel(x, pos_embed):
    mesh = plsc.VectorSubcoreMesh(core_axis_name="c", subcore_axis_name="s")
    k = functools.partial(
        pl.kernel,
        mesh=mesh,
        out_type=jax.ShapeDtypeStruct((S, B, D), jnp.float32),
        scratch_types=(
            [pltpu.VMEM((2, D), jnp.float32)]
            + [pltpu.VMEM((CROWS, D), jnp.float32) for _ in range(NBUF)]
            + [pltpu.SemaphoreType.DMA for _ in range(2 * NBUF + 1)]
        ),
    )(_sc_body)
    return k(x, pos_embed)
